# Initial kernel scaffold; baseline (speedup 1.0000x reference)
import jax
import jax.numpy as jnp
from jax.experimental import pallas as pl


def kernel(raw_ids, raw_values):
    raise NotImplementedError("write your pallas kernel here")


# speedup vs baseline: 2.3755x; 2.3755x over previous
import functools

import jax
import jax.numpy as jnp
from jax import lax
from jax.experimental import pallas as pl
from jax.experimental.pallas import tpu as pltpu
from jax.experimental.pallas import tpu_sc as plsc

R, C = 16384, 200
NEL = R * C
NC, NS = 2, 16
NW = NC * NS
PER_W = NEL // NW
CHUNK = 12800
NCHUNK = PER_W // CHUNK

_U = jnp.uint32
C1L, C1H = 0xED558CCD, 0xFF51AFD7
C2L, C2H = 0x1A85EC53, 0xC4CEB9FE


def _umulhi_parts(a, cl):
    y0, y1 = _U(cl & 0xFFFF), _U(cl >> 16)
    x0 = a & _U(0xFFFF)
    x1 = a >> _U(16)
    p00 = x0 * y0
    p10 = x1 * y0
    p01 = x0 * y1
    p11 = x1 * y1
    mid = p10 + (p00 >> _U(16))
    mid2 = p01 + (mid & _U(0xFFFF))
    return p11 + (mid >> _U(16)) + (mid2 >> _U(16))


def _hash16(al):
    hi = _umulhi_parts(al, C1L) + al * _U(C1H)
    lo = al * _U(C1L)
    lo = lo ^ (hi >> _U(1))
    hi2 = _umulhi_parts(lo, C2L) + lo * _U(C2H) + hi * _U(C2L)
    lo2 = lo * _U(C2L)
    lo2 = lo2 ^ (hi2 >> _U(1))
    s = ((lo2 & _U(0xFFFF))
         + (lo2 >> _U(16)) * _U(3036)
         + (hi2 & _U(0xFFFF)) * _U(14171)
         + (hi2 >> _U(16)) * _U(7531))
    si = s.astype(jnp.int32)
    qi = (si.astype(jnp.float32) * jnp.float32(1.0 / 15625.0)).astype(jnp.int32)
    r = si - qi * jnp.int32(15625)
    r = jnp.where(r < 0, r + jnp.int32(15625), r)
    r = jnp.where(r >= jnp.int32(15625), r - jnp.int32(15625), r)
    a6 = lo2 & _U(63)
    t = ((a6 - r.astype(jnp.uint32)) * _U(57)) & _U(63)
    return r + t.astype(jnp.int32) * jnp.int32(15625)


_MESH = plsc.VectorSubcoreMesh(
    core_axis_name="c", subcore_axis_name="s", num_cores=NC, num_subcores=NS)


@functools.partial(
    pl.kernel,
    out_type=jax.ShapeDtypeStruct((NEL,), jnp.int32),
    mesh=_MESH,
    scratch_types=[
        pltpu.VMEM((CHUNK,), jnp.int32),
        pltpu.VMEM((CHUNK,), jnp.int32),
    ],
)
def _sc_hash(ids_hbm, out_hbm, inbuf, outbuf):
    wid = lax.axis_index("s") * jnp.int32(NC) + lax.axis_index("c")
    base = wid * jnp.int32(PER_W)

    @pl.loop(jnp.int32(0), jnp.int32(NCHUNK))
    def _chunk(ci):
        off = base + ci * jnp.int32(CHUNK)
        pltpu.sync_copy(ids_hbm.at[pl.ds(off, CHUNK)], inbuf)

        @pl.loop(jnp.int32(0), jnp.int32(CHUNK // 16))
        def _body(j):
            j16 = j * jnp.int32(16)
            v = inbuf[pl.ds(j16, 16)]
            h = _hash16(plsc.bitcast(v, jnp.uint32))
            outbuf[pl.ds(j16, 16)] = h

        pltpu.sync_copy(outbuf, out_hbm.at[pl.ds(off, CHUNK)])


def kernel(raw_ids, raw_values):
    ids32 = raw_ids.astype(jnp.uint32).astype(jnp.int32).reshape(NEL)
    hashed = _sc_hash(ids32)
    return hashed.reshape(R, C).astype(jnp.int64), raw_values


# speedup vs baseline: 2.3942x; 1.0079x over previous
import functools

import jax
import jax.numpy as jnp
import numpy as np
from jax import lax
from jax.experimental import pallas as pl
from jax.experimental.pallas import tpu as pltpu
from jax.experimental.pallas import tpu_sc as plsc

R, C = 16384, 200
NEL = R * C
NC, NS = 2, 16
NW = NC * NS
PER_W = NEL // NW
CHUNK = 12800
NCHUNK = PER_W // CHUNK
UNROLL = 8

_U = jnp.uint32
C1L, C1H = 0xED558CCD, 0xFF51AFD7
C2L, C2H = 0x1A85EC53, 0xC4CEB9FE


def _umulhi_parts(a, cl):
    y0, y1 = _U(cl & 0xFFFF), _U(cl >> 16)
    x0 = a & _U(0xFFFF)
    x1 = a >> _U(16)
    p00 = x0 * y0
    p10 = x1 * y0
    p01 = x0 * y1
    p11 = x1 * y1
    mid = p10 + (p00 >> _U(16))
    mid2 = p01 + (mid & _U(0xFFFF))
    return p11 + (mid >> _U(16)) + (mid2 >> _U(16))


def _hash16(al):
    hi = _umulhi_parts(al, C1L) + al * _U(C1H)
    lo = al * _U(C1L)
    lo = lo ^ (hi >> _U(1))
    hi2 = _umulhi_parts(lo, C2L) + lo * _U(C2H) + hi * _U(C2L)
    lo2 = lo * _U(C2L)
    lo2 = lo2 ^ (hi2 >> _U(1))
    s = ((lo2 & _U(0xFFFF))
         + (lo2 >> _U(16)) * _U(3036)
         + (hi2 & _U(0xFFFF)) * _U(14171)
         + (hi2 >> _U(16)) * _U(7531))
    si = s.astype(jnp.int32)
    qi = (si.astype(jnp.float32) * jnp.float32(1.0 / 15625.0)).astype(jnp.int32)
    r = si - qi * jnp.int32(15625)
    r = jnp.where(r < 0, r + jnp.int32(15625), r)
    r = jnp.where(r >= jnp.int32(15625), r - jnp.int32(15625), r)
    a6 = lo2 & _U(63)
    t = ((a6 - r.astype(jnp.uint32)) * _U(57)) & _U(63)
    return r + t.astype(jnp.int32) * jnp.int32(15625)


_MESH = plsc.VectorSubcoreMesh(
    core_axis_name="c", subcore_axis_name="s", num_cores=NC, num_subcores=NS)


@functools.partial(
    pl.kernel,
    out_type=jax.ShapeDtypeStruct((NEL,), jnp.int32),
    mesh=_MESH,
    scratch_types=[
        pltpu.VMEM((CHUNK,), jnp.int32),
        pltpu.VMEM((CHUNK,), jnp.int32),
    ],
)
def _sc_hash(ids_hbm, out_hbm, inbuf, outbuf):
    wid = lax.axis_index("s") * jnp.int32(NC) + lax.axis_index("c")
    base = wid * jnp.int32(PER_W)

    @pl.loop(jnp.int32(0), jnp.int32(NCHUNK))
    def _chunk(ci):
        off = base + ci * jnp.int32(CHUNK)
        pltpu.sync_copy(ids_hbm.at[pl.ds(off, CHUNK)], inbuf)

        @pl.loop(jnp.int32(0), jnp.int32(CHUNK // (16 * UNROLL)))
        def _body(j):
            j16 = j * jnp.int32(16 * UNROLL)
            vs = [inbuf[pl.ds(j16 + jnp.int32(16 * u), 16)] for u in range(UNROLL)]
            hs = [_hash16(plsc.bitcast(v, jnp.uint32)) for v in vs]
            for u in range(UNROLL):
                outbuf[pl.ds(j16 + jnp.int32(16 * u), 16)] = hs[u]

        pltpu.sync_copy(outbuf, out_hbm.at[pl.ds(off, CHUNK)])


def kernel(raw_ids, raw_values):
    ids32 = raw_ids.astype(jnp.uint32).astype(jnp.int32).reshape(NEL)
    hashed = _sc_hash(ids32)
    return hashed.reshape(R, C).astype(jnp.int64), raw_values
